# baseline (speedup 1.0000x reference)
import functools

import jax
import jax.numpy as jnp
from jax import lax
from jax.experimental import pallas as pl
from jax.experimental.pallas import tpu as pltpu
from jax.experimental.pallas import tpu_sc as plsc

_ST_CHUNK = 16
_ST_NBUF = 4
_SP_CHUNK = 128
_SP_NBUF = 8


def _ring(in_copy, out_copy, nchunk, nbuf, lead):
    for j in range(min(lead, nchunk)):
        in_copy(j, j % nbuf).start()
    for i in range(nchunk):
        slot = i % nbuf
        in_copy(i, slot).wait()
        out_copy(i, slot).start()
        k = i + lead
        if k < nchunk:
            kslot = k % nbuf
            if k - nbuf >= 0:
                out_copy(k - nbuf, kslot).wait()
            in_copy(k, kslot).start()
    for i in range(max(0, nchunk - nbuf), nchunk):
        out_copy(i, i % nbuf).wait()


def _make_sc_copy(seq, dim, dtype):
    info = plsc.get_sparse_core_info()
    nc, ns = info.num_cores, info.num_subcores
    half = seq // nc
    rows_per_tile = 128
    st_rows = (ns - 1) * rows_per_tile
    sp_rows = half - st_rows
    st_nchunk = rows_per_tile // _ST_CHUNK
    sp_nchunk = sp_rows // _SP_CHUNK
    mesh = plsc.VectorSubcoreMesh(core_axis_name="c", subcore_axis_name="s")

    @functools.partial(
        pl.kernel,
        mesh=mesh,
        out_type=jax.ShapeDtypeStruct((1, seq, dim), dtype),
        scratch_types=[
            pltpu.VMEM((_ST_NBUF, _ST_CHUNK, dim), dtype),
            pltpu.VMEM_SHARED((_SP_NBUF, _SP_CHUNK, dim), dtype),
            pltpu.SemaphoreType.DMA((_ST_NBUF,)),
            pltpu.SemaphoreType.DMA((_ST_NBUF,)),
            pltpu.SemaphoreType.DMA((_SP_NBUF,)),
            pltpu.SemaphoreType.DMA((_SP_NBUF,)),
        ],
    )
    def sc_copy(w_hbm, out_hbm, tbuf, spbuf, st_in, st_out, sp_in, sp_out):
        cid = lax.axis_index("c")
        sid = lax.axis_index("s")
        scbase = cid * half
        tbase = scbase + sp_rows + jnp.maximum(sid - 1, 0) * rows_per_tile

        def st_in_copy(i, slot):
            return pltpu.make_async_copy(
                w_hbm.at[pl.ds(tbase + i * _ST_CHUNK, _ST_CHUNK)],
                tbuf.at[slot],
                st_in.at[slot],
            )

        def st_out_copy(i, slot):
            return pltpu.make_async_copy(
                tbuf.at[slot],
                out_hbm.at[0, pl.ds(tbase + i * _ST_CHUNK, _ST_CHUNK)],
                st_out.at[slot],
            )

        spbase = scbase

        def sp_in_copy(i, slot):
            return pltpu.make_async_copy(
                w_hbm.at[pl.ds(spbase + i * _SP_CHUNK, _SP_CHUNK)],
                spbuf.at[slot],
                sp_in.at[slot],
            )

        def sp_out_copy(i, slot):
            return pltpu.make_async_copy(
                spbuf.at[slot],
                out_hbm.at[0, pl.ds(spbase + i * _SP_CHUNK, _SP_CHUNK)],
                sp_out.at[slot],
            )

        @pl.when(sid == 0)
        def _():
            _ring(sp_in_copy, sp_out_copy, sp_nchunk, _SP_NBUF, _SP_NBUF - 1)

        @pl.when(sid > 0)
        def _():
            _ring(st_in_copy, st_out_copy, st_nchunk, _ST_NBUF, _ST_NBUF - 1)

    return sc_copy


def kernel(x, embedding_weight):
    seq = x.shape[1]
    dim = embedding_weight.shape[1]
    return _make_sc_copy(seq, dim, embedding_weight.dtype)(embedding_weight[:seq])


# speedup vs baseline: 1.0239x; 1.0239x over previous
import functools

import jax
import jax.numpy as jnp
from jax import lax
from jax.experimental import pallas as pl
from jax.experimental.pallas import tpu as pltpu
from jax.experimental.pallas import tpu_sc as plsc

_CHUNK_ROWS = 128
_NBUF = 8
_IN_LEAD = 7


def _make_sc_copy(seq, dim, dtype):
    info = plsc.get_sparse_core_info()
    nc = info.num_cores
    rows_per_w = seq // nc
    chunk = _CHUNK_ROWS
    nbuf = _NBUF
    nchunk = rows_per_w // chunk
    mesh = plsc.ScalarSubcoreMesh(axis_name="c", num_cores=nc)

    @functools.partial(
        pl.kernel,
        mesh=mesh,
        out_type=jax.ShapeDtypeStruct((1, seq, dim), dtype),
        scratch_types=[
            pltpu.VMEM_SHARED((nbuf, chunk, dim), dtype),
            pltpu.SemaphoreType.DMA((nbuf,)),
            pltpu.SemaphoreType.DMA((nbuf,)),
        ],
    )
    def sc_copy(w_hbm, out_hbm, buf, in_sems, out_sems):
        base = lax.axis_index("c") * rows_per_w

        def in_copy(i, slot):
            return pltpu.make_async_copy(
                w_hbm.at[pl.ds(base + i * chunk, chunk)],
                buf.at[slot],
                in_sems.at[slot],
            )

        def out_copy(i, slot):
            return pltpu.make_async_copy(
                buf.at[slot],
                out_hbm.at[0, pl.ds(base + i * chunk, chunk)],
                out_sems.at[slot],
            )

        lead = _IN_LEAD
        for j in range(min(lead, nchunk)):
            in_copy(j, j % nbuf).start()
        for i in range(nchunk):
            slot = i % nbuf
            in_copy(i, slot).wait()
            out_copy(i, slot).start()
            k = i + lead
            if k < nchunk:
                kslot = k % nbuf
                if k - nbuf >= 0:
                    out_copy(k - nbuf, kslot).wait()
                in_copy(k, kslot).start()
        for i in range(max(0, nchunk - nbuf), nchunk):
            out_copy(i, i % nbuf).wait()

    return sc_copy


def kernel(x, embedding_weight):
    seq = x.shape[1]
    dim = embedding_weight.shape[1]
    return _make_sc_copy(seq, dim, embedding_weight.dtype)(embedding_weight[:seq])
